# Initial kernel scaffold; baseline (speedup 1.0000x reference)
import jax
import jax.numpy as jnp
from jax import lax
from jax.experimental import pallas as pl
from jax.experimental.pallas import tpu as pltpu

NTOK = 16384
D = 4096
SQRT_K = 8
NE = SQRT_K * SQRT_K
TOP_K = 8
BLK = 512


def _router_body(log_tau_ref, x_ref, wct_ref, idx_ref, gates_ref, scores_ref):
    row = lax.broadcasted_iota(jnp.int32, (2 * SQRT_K, NE), 0)
    col = lax.broadcasted_iota(jnp.int32, (2 * SQRT_K, NE), 1)
    e1 = (row < SQRT_K) & ((col // SQRT_K) == row)
    e2 = (row >= SQRT_K) & ((col % SQRT_K) == (row - SQRT_K))
    expand = (e1 | e2).astype(jnp.float32)

    s = jnp.dot(x_ref[...], wct_ref[...], preferred_element_type=jnp.float32)
    scores = jnp.dot(s, expand, preferred_element_type=jnp.float32)
    scores_ref[...] = scores

    tau = jnp.exp(log_tau_ref[0, 0])
    lane = lax.broadcasted_iota(jnp.int32, (BLK, NE), 1)
    col8 = lax.broadcasted_iota(jnp.int32, (BLK, TOP_K), 1)
    work = scores
    vals8 = jnp.zeros((BLK, TOP_K), jnp.float32)
    idx8 = jnp.zeros((BLK, TOP_K), jnp.int32)
    for k in range(TOP_K):
        m = jnp.max(work, axis=1, keepdims=True)
        pick = jnp.min(jnp.where(work == m, lane, NE), axis=1, keepdims=True)
        vals8 = jnp.where(col8 == k, m, vals8)
        idx8 = jnp.where(col8 == k, pick, idx8)
        work = jnp.where(lane == pick, -jnp.inf, work)

    mx = jnp.max(vals8, axis=1, keepdims=True)
    ex = jnp.exp((vals8 - mx) / tau)
    gates_ref[...] = ex / jnp.sum(ex, axis=1, keepdims=True)
    idx_ref[...] = idx8


@jax.jit
def kernel(x, W1, W2, log_tau):
    wct = jnp.concatenate([W1, W2], axis=0).T
    lt = log_tau.reshape(1, 1)
    grid = NTOK // BLK
    idx, gates, scores = pl.pallas_call(
        _router_body,
        grid=(grid,),
        in_specs=[
            pl.BlockSpec(memory_space=pltpu.SMEM),
            pl.BlockSpec((BLK, D), lambda i: (i, 0)),
            pl.BlockSpec((D, 2 * SQRT_K), lambda i: (0, 0)),
        ],
        out_specs=[
            pl.BlockSpec((BLK, TOP_K), lambda i: (i, 0)),
            pl.BlockSpec((BLK, TOP_K), lambda i: (i, 0)),
            pl.BlockSpec((BLK, NE), lambda i: (i, 0)),
        ],
        out_shape=[
            jax.ShapeDtypeStruct((NTOK, TOP_K), jnp.int32),
            jax.ShapeDtypeStruct((NTOK, TOP_K), jnp.float32),
            jax.ShapeDtypeStruct((NTOK, NE), jnp.float32),
        ],
    )(lt, x, wct)
    return idx, gates, scores


# speedup vs baseline: 1.2607x; 1.2607x over previous
import jax
import jax.numpy as jnp
from jax import lax
from jax.experimental import pallas as pl
from jax.experimental.pallas import tpu as pltpu

NTOK = 16384
D = 4096
SQRT_K = 8
NE = SQRT_K * SQRT_K
TOP_K = 8
BLK = 512


def _router_body(log_tau_ref, x_ref, wct_ref, idx_ref, gates_ref, scores_ref):
    s = jnp.dot(
        x_ref[...].astype(jnp.bfloat16),
        wct_ref[...].astype(jnp.bfloat16),
        preferred_element_type=jnp.float32,
    )
    s1 = s[:, :SQRT_K]
    s2 = s[:, SQRT_K:]
    rep1 = jnp.concatenate(
        [jnp.broadcast_to(s1[:, i : i + 1], (BLK, SQRT_K)) for i in range(SQRT_K)],
        axis=1,
    )
    tile2 = jnp.concatenate([s2] * SQRT_K, axis=1)
    scores = rep1 + tile2
    scores_ref[...] = scores

    tau = jnp.exp(log_tau_ref[0, 0])
    lane = lax.broadcasted_iota(jnp.int32, (BLK, NE), 1)
    col8 = lax.broadcasted_iota(jnp.int32, (BLK, TOP_K), 1)
    work = scores
    vals8 = jnp.zeros((BLK, TOP_K), jnp.float32)
    idx8 = jnp.zeros((BLK, TOP_K), jnp.int32)
    for k in range(TOP_K):
        m = jnp.max(work, axis=1, keepdims=True)
        pick = jnp.min(jnp.where(work == m, lane, NE), axis=1, keepdims=True)
        vals8 = jnp.where(col8 == k, m, vals8)
        idx8 = jnp.where(col8 == k, pick, idx8)
        work = jnp.where(lane == pick, -jnp.inf, work)

    mx = jnp.max(vals8, axis=1, keepdims=True)
    ex = jnp.exp((vals8 - mx) / tau)
    gates_ref[...] = ex / jnp.sum(ex, axis=1, keepdims=True)
    idx_ref[...] = idx8


@jax.jit
def kernel(x, W1, W2, log_tau):
    wct = jnp.concatenate([W1, W2], axis=0).T
    lt = log_tau.reshape(1, 1)
    grid = NTOK // BLK
    idx, gates, scores = pl.pallas_call(
        _router_body,
        grid=(grid,),
        in_specs=[
            pl.BlockSpec(memory_space=pltpu.SMEM),
            pl.BlockSpec((BLK, D), lambda i: (i, 0)),
            pl.BlockSpec((D, 2 * SQRT_K), lambda i: (0, 0)),
        ],
        out_specs=[
            pl.BlockSpec((BLK, TOP_K), lambda i: (i, 0)),
            pl.BlockSpec((BLK, TOP_K), lambda i: (i, 0)),
            pl.BlockSpec((BLK, NE), lambda i: (i, 0)),
        ],
        out_shape=[
            jax.ShapeDtypeStruct((NTOK, TOP_K), jnp.int32),
            jax.ShapeDtypeStruct((NTOK, TOP_K), jnp.float32),
            jax.ShapeDtypeStruct((NTOK, NE), jnp.float32),
        ],
    )(lt, x, wct)
    return idx, gates, scores


# speedup vs baseline: 1.5348x; 1.2174x over previous
import jax
import jax.numpy as jnp
from jax import lax
from jax.experimental import pallas as pl
from jax.experimental.pallas import tpu as pltpu

NTOK = 16384
D = 4096
SQRT_K = 8
NE = SQRT_K * SQRT_K
TOP_K = 8
BLK = 512


def _router_body(log_tau_ref, x_ref, wct_ref, idx_ref, gates_ref, scores_ref):
    s = jnp.dot(
        x_ref[...].astype(jnp.bfloat16),
        wct_ref[...].astype(jnp.bfloat16),
        preferred_element_type=jnp.float32,
    )
    row = lax.broadcasted_iota(jnp.int32, (2 * SQRT_K, NE), 0)
    col = lax.broadcasted_iota(jnp.int32, (2 * SQRT_K, NE), 1)
    exp1 = ((row < SQRT_K) & ((col // SQRT_K) == row)).astype(jnp.float32)
    exp2 = ((row >= SQRT_K) & ((col % SQRT_K) == (row - SQRT_K))).astype(
        jnp.float32
    )
    rep1 = jnp.dot(s, exp1, preferred_element_type=jnp.float32,
                   precision=lax.Precision.HIGHEST)
    tile2 = jnp.dot(s, exp2, preferred_element_type=jnp.float32,
                    precision=lax.Precision.HIGHEST)
    scores = rep1 + tile2
    scores_ref[...] = scores

    tau = jnp.exp(log_tau_ref[0, 0])
    lane = lax.broadcasted_iota(jnp.int32, (BLK, NE), 1).astype(jnp.float32)
    col8 = lax.broadcasted_iota(jnp.int32, (BLK, TOP_K), 1)
    work = scores
    vals8 = jnp.zeros((BLK, TOP_K), jnp.float32)
    idx8 = jnp.zeros((BLK, TOP_K), jnp.float32)
    for k in range(TOP_K):
        m = jnp.max(work, axis=1, keepdims=True)
        pick = jnp.min(jnp.where(work == m, lane, jnp.float32(NE)), axis=1,
                       keepdims=True)
        vals8 = jnp.where(col8 == k, m, vals8)
        idx8 = jnp.where(col8 == k, pick, idx8)
        work = jnp.where(lane == pick, -jnp.inf, work)

    mx = jnp.max(vals8, axis=1, keepdims=True)
    ex = jnp.exp((vals8 - mx) / tau)
    gates_ref[...] = ex / jnp.sum(ex, axis=1, keepdims=True)
    idx_ref[...] = idx8.astype(jnp.int32)


@jax.jit
def kernel(x, W1, W2, log_tau):
    wct = jnp.concatenate([W1, W2], axis=0).T
    lt = log_tau.reshape(1, 1)
    grid = NTOK // BLK
    idx, gates, scores = pl.pallas_call(
        _router_body,
        grid=(grid,),
        in_specs=[
            pl.BlockSpec(memory_space=pltpu.SMEM),
            pl.BlockSpec((BLK, D), lambda i: (i, 0)),
            pl.BlockSpec((D, 2 * SQRT_K), lambda i: (0, 0)),
        ],
        out_specs=[
            pl.BlockSpec((BLK, TOP_K), lambda i: (i, 0)),
            pl.BlockSpec((BLK, TOP_K), lambda i: (i, 0)),
            pl.BlockSpec((BLK, NE), lambda i: (i, 0)),
        ],
        out_shape=[
            jax.ShapeDtypeStruct((NTOK, TOP_K), jnp.int32),
            jax.ShapeDtypeStruct((NTOK, TOP_K), jnp.float32),
            jax.ShapeDtypeStruct((NTOK, NE), jnp.float32),
        ],
    )(lt, x, wct)
    return idx, gates, scores


# speedup vs baseline: 1.6388x; 1.0678x over previous
import jax
import jax.numpy as jnp
from jax import lax
from jax.experimental import pallas as pl
from jax.experimental.pallas import tpu as pltpu

NTOK = 16384
D = 4096
SQRT_K = 8
NE = SQRT_K * SQRT_K
TOP_K = 8
BLK = 1024


def _router_body(log_tau_ref, x_ref, wct_ref, idx_ref, gates_ref, scores_ref):
    s = jnp.dot(
        x_ref[...].astype(jnp.bfloat16),
        wct_ref[...].astype(jnp.bfloat16),
        preferred_element_type=jnp.float32,
    )
    row = lax.broadcasted_iota(jnp.int32, (2 * SQRT_K, NE), 0)
    col = lax.broadcasted_iota(jnp.int32, (2 * SQRT_K, NE), 1)
    exp1 = ((row < SQRT_K) & ((col // SQRT_K) == row)).astype(jnp.float32)
    exp2 = ((row >= SQRT_K) & ((col % SQRT_K) == (row - SQRT_K))).astype(
        jnp.float32
    )
    rep1 = jnp.dot(s, exp1, preferred_element_type=jnp.float32,
                   precision=lax.Precision.HIGHEST)
    tile2 = jnp.dot(s, exp2, preferred_element_type=jnp.float32,
                    precision=lax.Precision.HIGHEST)
    scores = rep1 + tile2
    scores_ref[...] = scores

    tau = jnp.exp(log_tau_ref[0, 0])
    lane = lax.broadcasted_iota(jnp.int32, (BLK, NE), 1).astype(jnp.float32)
    col8 = lax.broadcasted_iota(jnp.int32, (BLK, TOP_K), 1)
    work = scores
    vals8 = jnp.zeros((BLK, TOP_K), jnp.float32)
    idx8 = jnp.zeros((BLK, TOP_K), jnp.float32)
    for k in range(TOP_K):
        m = jnp.max(work, axis=1, keepdims=True)
        pick = jnp.min(jnp.where(work == m, lane, jnp.float32(NE)), axis=1,
                       keepdims=True)
        vals8 = jnp.where(col8 == k, m, vals8)
        idx8 = jnp.where(col8 == k, pick, idx8)
        work = jnp.where(lane == pick, -jnp.inf, work)

    mx = jnp.max(vals8, axis=1, keepdims=True)
    ex = jnp.exp((vals8 - mx) / tau)
    gates_ref[...] = ex / jnp.sum(ex, axis=1, keepdims=True)
    idx_ref[...] = idx8.astype(jnp.int32)


@jax.jit
def kernel(x, W1, W2, log_tau):
    wct = jnp.concatenate([W1, W2], axis=0).T
    lt = log_tau.reshape(1, 1)
    grid = NTOK // BLK
    idx, gates, scores = pl.pallas_call(
        _router_body,
        grid=(grid,),
        in_specs=[
            pl.BlockSpec(memory_space=pltpu.SMEM),
            pl.BlockSpec((BLK, D), lambda i: (i, 0)),
            pl.BlockSpec((D, 2 * SQRT_K), lambda i: (0, 0)),
        ],
        out_specs=[
            pl.BlockSpec((BLK, TOP_K), lambda i: (i, 0)),
            pl.BlockSpec((BLK, TOP_K), lambda i: (i, 0)),
            pl.BlockSpec((BLK, NE), lambda i: (i, 0)),
        ],
        out_shape=[
            jax.ShapeDtypeStruct((NTOK, TOP_K), jnp.int32),
            jax.ShapeDtypeStruct((NTOK, TOP_K), jnp.float32),
            jax.ShapeDtypeStruct((NTOK, NE), jnp.float32),
        ],
    )(lt, x, wct)
    return idx, gates, scores
